# Initial kernel scaffold; baseline (speedup 1.0000x reference)
import jax
import jax.numpy as jnp
from jax.experimental import pallas as pl


def kernel(x, c, edge_index, batch, W1e, b1e, W2e, b2e, W1c, b1c, W2c, b2c, Wm, bm, Wp1, bp1, Wp2, bp2, eps):
    raise NotImplementedError("write your pallas kernel here")


# speedup vs baseline: 3.0224x; 3.0224x over previous
import functools

import jax
import jax.numpy as jnp
from jax.experimental import pallas as pl
from jax.experimental.pallas import tpu as pltpu
from jax.experimental.pallas import tpu_sc as plsc

_N = 10000
_F = 128
_E = 320000
_G = 64
_C = 10

_NW = 32
_K = 128
_CH = 80
_EPW = _CH * _K
_EPAD = _NW * _EPW
_ACC_ROWS = 10240
_RPS = _ACC_ROWS // 16
_ZR = 80
_TRASH = 10200

_B = 400
_NB = _N // _B

_mesh = plsc.VectorSubcoreMesh(core_axis_name="c", subcore_axis_name="s")


@functools.partial(
    pl.kernel,
    mesh=_mesh,
    out_type=jax.ShapeDtypeStruct((2, _ACC_ROWS, _F), jnp.float32),
    scratch_types=[
        pltpu.VMEM((_CH, _K), jnp.int32),
        pltpu.VMEM((_CH, _K), jnp.int32),
        pltpu.VMEM((_K, _F), jnp.float32),
        pltpu.VMEM((_ZR, _F), jnp.float32),
        pltpu.VMEM_SHARED((_ACC_ROWS, _F), jnp.float32),
        pltpu.SemaphoreType.DMA,
    ],
)
def _edge_scatter(table, srcs, dsts, out, src_v, dst_v, gbuf, zbuf, acc, sem):
    c = jax.lax.axis_index("c")
    s = jax.lax.axis_index("s")
    wid = s * 2 + c

    def _z(i, carry):
        r = i // 8
        k = i % 8
        zbuf[r, pl.ds(k * 16, 16)] = jnp.zeros((16,), jnp.float32)
        return carry

    jax.lax.fori_loop(0, _ZR * 8, _z, 0)

    def _za(t, carry):
        pltpu.sync_copy(zbuf, acc.at[pl.ds(s * _RPS + t * _ZR, _ZR)])
        return carry

    jax.lax.fori_loop(0, _RPS // _ZR, _za, 0)

    pltpu.sync_copy(srcs.at[wid], src_v)
    pltpu.sync_copy(dsts.at[wid], dst_v)

    plsc.subcore_barrier()

    def _chunk(j, carry):
        pltpu.async_copy(table.at[src_v.at[j]], gbuf, sem).wait()
        pltpu.sync_copy(gbuf, acc.at[dst_v.at[j]], add=True)
        return carry

    jax.lax.fori_loop(0, _CH, _chunk, 0)

    plsc.subcore_barrier()

    pltpu.sync_copy(acc.at[pl.ds(s * _RPS, _RPS)],
                    out.at[c, pl.ds(s * _RPS, _RPS)])


def _lin_body(scale_ref, h_ref, p_ref, w_ref, b_ref, o_ref):
    a = h_ref[...] * scale_ref[0, 0] + p_ref[0] + p_ref[1]
    y = jnp.dot(a, w_ref[...], preferred_element_type=jnp.float32) + b_ref[...]
    o_ref[...] = jnp.where(y >= 0.0, y, 0.2 * y)


def _linear(h, p, W, b, scale):
    return pl.pallas_call(
        _lin_body,
        grid=(_NB,),
        in_specs=[
            pl.BlockSpec(memory_space=pltpu.SMEM),
            pl.BlockSpec((_B, _F), lambda i: (i, 0)),
            pl.BlockSpec((2, _B, _F), lambda i: (0, i, 0)),
            pl.BlockSpec((_F, _F), lambda i: (0, 0)),
            pl.BlockSpec((1, _F), lambda i: (0, 0)),
        ],
        out_specs=pl.BlockSpec((_B, _F), lambda i: (i, 0)),
        out_shape=jax.ShapeDtypeStruct((_N, _F), jnp.float32),
    )(scale, h, p, W, b)


def _merge_body(xe_ref, xc_ref, we_ref, wc_ref, b_ref, o_ref):
    y = jnp.dot(xe_ref[...], we_ref[...], preferred_element_type=jnp.float32)
    y = y + jnp.dot(xc_ref[...], wc_ref[...], preferred_element_type=jnp.float32)
    y = y + b_ref[...]
    o_ref[...] = jnp.where(y >= 0.0, y, 0.2 * y)


def _merge(xe, xc, We, Wc, b):
    return pl.pallas_call(
        _merge_body,
        grid=(_NB,),
        in_specs=[
            pl.BlockSpec((_B, _F), lambda i: (i, 0)),
            pl.BlockSpec((_B, _F), lambda i: (i, 0)),
            pl.BlockSpec((_F, _F), lambda i: (0, 0)),
            pl.BlockSpec((_F, _F), lambda i: (0, 0)),
            pl.BlockSpec((1, _F), lambda i: (0, 0)),
        ],
        out_specs=pl.BlockSpec((_B, _F), lambda i: (i, 0)),
        out_shape=jax.ShapeDtypeStruct((_N, _F), jnp.float32),
    )(xe, xc, We, Wc, b)


def _head_body(batch_ref, h_ref, wp1_ref, bp1_ref, wp2_ref, bp2_ref,
               gr_ref, lo_ref, cnt_ref):
    i = pl.program_id(0)

    @pl.when(i == 0)
    def _():
        gr_ref[...] = jnp.zeros_like(gr_ref)
        lo_ref[...] = jnp.zeros_like(lo_ref)
        cnt_ref[...] = jnp.zeros_like(cnt_ref)

    b_row = batch_ref[0, 0, :]
    seg = jax.lax.broadcasted_iota(jnp.int32, (_G, _B), 0)
    P = (seg == b_row[None, :]).astype(jnp.float32)

    h = h_ref[...]
    gr_ref[...] += jnp.dot(P, h, preferred_element_type=jnp.float32)
    t = jnp.dot(h, wp1_ref[...], preferred_element_type=jnp.float32) + bp1_ref[...]
    t = jnp.maximum(t, 0.0)
    lg = jnp.dot(t, wp2_ref[...], preferred_element_type=jnp.float32) + bp2_ref[...]
    lo_ref[...] += jnp.dot(P, lg, preferred_element_type=jnp.float32)
    cnt_ref[...] += jnp.broadcast_to(jnp.sum(P, axis=1, keepdims=True), (_G, _F))

    @pl.when(i == _NB - 1)
    def _():
        cm = jnp.maximum(cnt_ref[...], 1.0)
        gr_ref[...] = gr_ref[...] / cm
        lo_ref[...] = lo_ref[...] / cm


def _head(batch3, h, Wp1, bp1, Wp2p, bp2p):
    return pl.pallas_call(
        _head_body,
        grid=(_NB,),
        in_specs=[
            pl.BlockSpec((1, 1, _B), lambda i: (i, 0, 0)),
            pl.BlockSpec((_B, _F), lambda i: (i, 0)),
            pl.BlockSpec((_F, _F), lambda i: (0, 0)),
            pl.BlockSpec((1, _F), lambda i: (0, 0)),
            pl.BlockSpec((_F, _F), lambda i: (0, 0)),
            pl.BlockSpec((1, _F), lambda i: (0, 0)),
        ],
        out_specs=[
            pl.BlockSpec((_G, _F), lambda i: (0, 0)),
            pl.BlockSpec((_G, _F), lambda i: (0, 0)),
        ],
        out_shape=[
            jax.ShapeDtypeStruct((_G, _F), jnp.float32),
            jax.ShapeDtypeStruct((_G, _F), jnp.float32),
        ],
        scratch_shapes=[pltpu.VMEM((_G, _F), jnp.float32)],
    )(batch3, h, Wp1, bp1, Wp2p, bp2p)


def kernel(x, c, edge_index, batch, W1e, b1e, W2e, b2e, W1c, b1c, W2c, b2c,
           Wm, bm, Wp1, bp1, Wp2, bp2, eps):
    pad = _EPAD - _E
    src_p = jnp.concatenate(
        [edge_index[0], jnp.zeros((pad,), jnp.int32)]).reshape(_NW, _CH, _K)
    dst_p = jnp.concatenate(
        [edge_index[1], jnp.full((pad,), _TRASH, jnp.int32)]).reshape(_NW, _CH, _K)

    scales = (2.0 + eps).astype(jnp.float32)

    def gin(h, W, b, k):
        p = _edge_scatter(h, src_p, dst_p)
        return _linear(h, p, W, b.reshape(1, _F), scales[k].reshape(1, 1))

    xe = gin(x, W1e, b1e, 0)
    xe = gin(xe, W2e, b2e, 1)
    xc = gin(c, W1c, b1c, 2)
    xc = gin(xc, W2c, b2c, 3)

    h = _merge(xe, xc, Wm[:_F], Wm[_F:], bm.reshape(1, _F))

    batch3 = batch.reshape(_NB, 1, _B)
    Wp2p = jnp.zeros((_F, _F), jnp.float32).at[:, :_C].set(Wp2)
    bp2p = jnp.zeros((1, _F), jnp.float32).at[0, :_C].set(bp2)

    gr, lo = _head(batch3, h, Wp1, bp1.reshape(1, _F), Wp2p, bp2p)
    return gr, lo[:, :_C]


# speedup vs baseline: 3.6698x; 1.2142x over previous
import functools

import jax
import jax.numpy as jnp
from jax.experimental import pallas as pl
from jax.experimental.pallas import tpu as pltpu
from jax.experimental.pallas import tpu_sc as plsc

_N = 10000
_F = 128
_E = 320000
_G = 64
_C = 10

_NW = 32
_K = 128
_CH = 80
_EPW = _CH * _K
_EPAD = _NW * _EPW
_PCH = 40
_ACC_ROWS = 10112
_RPS = _ACC_ROWS // 16
_ZR = 8
_TRASH = 10050

_B = 400
_NB = _N // _B

_mesh = plsc.VectorSubcoreMesh(core_axis_name="c", subcore_axis_name="s")


@functools.partial(
    pl.kernel,
    mesh=_mesh,
    out_type=jax.ShapeDtypeStruct((2, _ACC_ROWS, _F), jnp.float32),
    scratch_types=[
        pltpu.VMEM((_PCH, _K), jnp.int32),
        pltpu.VMEM((_PCH, _K), jnp.int32),
        pltpu.VMEM((_K, _F), jnp.float32),
        pltpu.VMEM((_K, _F), jnp.float32),
        pltpu.VMEM((_ZR, _F), jnp.float32),
        pltpu.VMEM_SHARED((_ACC_ROWS, _F), jnp.float32),
        pltpu.SemaphoreType.DMA,
        pltpu.SemaphoreType.DMA,
    ],
)
def _edge_scatter(table, srcs, dsts, out, src_v, dst_v, gbufa, gbufb, zbuf,
                  acc, sema, semb):
    c = jax.lax.axis_index("c")
    s = jax.lax.axis_index("s")
    wid = s * 2 + c

    def _z(i, carry):
        r = i // 8
        k = i % 8
        zbuf[r, pl.ds(k * 16, 16)] = jnp.zeros((16,), jnp.float32)
        return carry

    jax.lax.fori_loop(0, _ZR * 8, _z, 0)

    def _za(t, carry):
        pltpu.sync_copy(zbuf, acc.at[pl.ds(s * _RPS + t * _ZR, _ZR)])
        return carry

    jax.lax.fori_loop(0, _RPS // _ZR, _za, 0)

    plsc.subcore_barrier()

    for p in range(_CH // _PCH):
        pltpu.sync_copy(srcs.at[wid, pl.ds(p * _PCH, _PCH)], src_v)
        pltpu.sync_copy(dsts.at[wid, pl.ds(p * _PCH, _PCH)], dst_v)
        pltpu.async_copy(table.at[src_v.at[0]], gbufa, sema)

        def _pair(t, carry):
            j = t * 2
            pltpu.async_copy(table.at[src_v.at[j + 1]], gbufb, semb)
            pltpu.make_async_copy(table.at[src_v.at[j]], gbufa, sema).wait()
            pltpu.sync_copy(gbufa, acc.at[dst_v.at[j]], add=True)

            @pl.when(t < _PCH // 2 - 1)
            def _():
                pltpu.async_copy(table.at[src_v.at[j + 2]], gbufa, sema)

            pltpu.make_async_copy(table.at[src_v.at[j + 1]], gbufb, semb).wait()
            pltpu.sync_copy(gbufb, acc.at[dst_v.at[j + 1]], add=True)
            return carry

        jax.lax.fori_loop(0, _PCH // 2, _pair, 0)

    plsc.subcore_barrier()

    pltpu.sync_copy(acc.at[pl.ds(s * _RPS, _RPS)],
                    out.at[c, pl.ds(s * _RPS, _RPS)])


def _lin_body(scale_ref, h_ref, p_ref, w_ref, b_ref, o_ref):
    a = h_ref[...] * scale_ref[0, 0] + p_ref[0] + p_ref[1]
    y = jnp.dot(a, w_ref[...], preferred_element_type=jnp.float32) + b_ref[...]
    o_ref[...] = jnp.where(y >= 0.0, y, 0.2 * y)


def _linear(h, p, W, b, scale):
    return pl.pallas_call(
        _lin_body,
        grid=(_NB,),
        in_specs=[
            pl.BlockSpec(memory_space=pltpu.SMEM),
            pl.BlockSpec((_B, _F), lambda i: (i, 0)),
            pl.BlockSpec((2, _B, _F), lambda i: (0, i, 0)),
            pl.BlockSpec((_F, _F), lambda i: (0, 0)),
            pl.BlockSpec((1, _F), lambda i: (0, 0)),
        ],
        out_specs=pl.BlockSpec((_B, _F), lambda i: (i, 0)),
        out_shape=jax.ShapeDtypeStruct((_N, _F), jnp.float32),
    )(scale, h, p, W, b)


def _merge_body(xe_ref, xc_ref, we_ref, wc_ref, b_ref, o_ref):
    y = jnp.dot(xe_ref[...], we_ref[...], preferred_element_type=jnp.float32)
    y = y + jnp.dot(xc_ref[...], wc_ref[...], preferred_element_type=jnp.float32)
    y = y + b_ref[...]
    o_ref[...] = jnp.where(y >= 0.0, y, 0.2 * y)


def _merge(xe, xc, We, Wc, b):
    return pl.pallas_call(
        _merge_body,
        grid=(_NB,),
        in_specs=[
            pl.BlockSpec((_B, _F), lambda i: (i, 0)),
            pl.BlockSpec((_B, _F), lambda i: (i, 0)),
            pl.BlockSpec((_F, _F), lambda i: (0, 0)),
            pl.BlockSpec((_F, _F), lambda i: (0, 0)),
            pl.BlockSpec((1, _F), lambda i: (0, 0)),
        ],
        out_specs=pl.BlockSpec((_B, _F), lambda i: (i, 0)),
        out_shape=jax.ShapeDtypeStruct((_N, _F), jnp.float32),
    )(xe, xc, We, Wc, b)


def _head_body(batch_ref, h_ref, wp1_ref, bp1_ref, wp2_ref, bp2_ref,
               gr_ref, lo_ref, cnt_ref):
    i = pl.program_id(0)

    @pl.when(i == 0)
    def _():
        gr_ref[...] = jnp.zeros_like(gr_ref)
        lo_ref[...] = jnp.zeros_like(lo_ref)
        cnt_ref[...] = jnp.zeros_like(cnt_ref)

    b_row = batch_ref[0, 0, :]
    seg = jax.lax.broadcasted_iota(jnp.int32, (_G, _B), 0)
    P = (seg == b_row[None, :]).astype(jnp.float32)

    h = h_ref[...]
    gr_ref[...] += jnp.dot(P, h, preferred_element_type=jnp.float32)
    t = jnp.dot(h, wp1_ref[...], preferred_element_type=jnp.float32) + bp1_ref[...]
    t = jnp.maximum(t, 0.0)
    lg = jnp.dot(t, wp2_ref[...], preferred_element_type=jnp.float32) + bp2_ref[...]
    lo_ref[...] += jnp.dot(P, lg, preferred_element_type=jnp.float32)
    cnt_ref[...] += jnp.broadcast_to(jnp.sum(P, axis=1, keepdims=True), (_G, _F))

    @pl.when(i == _NB - 1)
    def _():
        cm = jnp.maximum(cnt_ref[...], 1.0)
        gr_ref[...] = gr_ref[...] / cm
        lo_ref[...] = lo_ref[...] / cm


def _head(batch3, h, Wp1, bp1, Wp2p, bp2p):
    return pl.pallas_call(
        _head_body,
        grid=(_NB,),
        in_specs=[
            pl.BlockSpec((1, 1, _B), lambda i: (i, 0, 0)),
            pl.BlockSpec((_B, _F), lambda i: (i, 0)),
            pl.BlockSpec((_F, _F), lambda i: (0, 0)),
            pl.BlockSpec((1, _F), lambda i: (0, 0)),
            pl.BlockSpec((_F, _F), lambda i: (0, 0)),
            pl.BlockSpec((1, _F), lambda i: (0, 0)),
        ],
        out_specs=[
            pl.BlockSpec((_G, _F), lambda i: (0, 0)),
            pl.BlockSpec((_G, _F), lambda i: (0, 0)),
        ],
        out_shape=[
            jax.ShapeDtypeStruct((_G, _F), jnp.float32),
            jax.ShapeDtypeStruct((_G, _F), jnp.float32),
        ],
        scratch_shapes=[pltpu.VMEM((_G, _F), jnp.float32)],
    )(batch3, h, Wp1, bp1, Wp2p, bp2p)


def kernel(x, c, edge_index, batch, W1e, b1e, W2e, b2e, W1c, b1c, W2c, b2c,
           Wm, bm, Wp1, bp1, Wp2, bp2, eps):
    pad = _EPAD - _E
    src_p = jnp.concatenate(
        [edge_index[0], jnp.zeros((pad,), jnp.int32)]).reshape(_NW, _CH, _K)
    dst_p = jnp.concatenate(
        [edge_index[1], jnp.full((pad,), _TRASH, jnp.int32)]).reshape(_NW, _CH, _K)

    scales = (2.0 + eps).astype(jnp.float32)

    def gin(h, W, b, k):
        p = _edge_scatter(h, src_p, dst_p)
        return _linear(h, p, W, b.reshape(1, _F), scales[k].reshape(1, 1))

    xe = gin(x, W1e, b1e, 0)
    xe = gin(xe, W2e, b2e, 1)
    xc = gin(c, W1c, b1c, 2)
    xc = gin(xc, W2c, b2c, 3)

    h = _merge(xe, xc, Wm[:_F], Wm[_F:], bm.reshape(1, _F))

    batch3 = batch.reshape(_NB, 1, _B)
    Wp2p = jnp.zeros((_F, _F), jnp.float32).at[:, :_C].set(Wp2)
    bp2p = jnp.zeros((1, _F), jnp.float32).at[0, :_C].set(bp2)

    gr, lo = _head(batch3, h, Wp1, bp1.reshape(1, _F), Wp2p, bp2p)
    return gr, lo[:, :_C]
